# baseline (speedup 1.0000x reference)
import jax
import jax.numpy as jnp
from jax import lax
from jax.experimental import pallas as pl
from jax.experimental.pallas import tpu as pltpu
from jax.experimental.pallas import tpu_sc as plsc

VOCAB = 1000000
EMBED_DIM = 32
BATCH = 16384
HIST = 50

HIST_PAD = 56
D_PAD = 128

NC = 2
NS = 16
NW = NC * NS

B_TOTAL = BATCH * HIST
BATCH_PER_W = BATCH // NW
CB = 32
CHUNK = CB * HIST
N_CHUNKS = BATCH_PER_W // CB


def _gather_body(idx_hbm, table_hbm, out_hbm, idx_v, rows_v, sg0, sg1, sw0, sw1):
    wid = lax.axis_index("s") * NC + lax.axis_index("c")
    b_base = wid * BATCH_PER_W
    sg = (sg0, sg1)
    sw = (sw0, sw1)

    def load_idx(g, s):
        off = (b_base + g * CB) * HIST
        pltpu.sync_copy(idx_hbm.at[pl.ds(off, CHUNK)], idx_v.at[s])

    def start_gather(s):
        pltpu.async_copy(table_hbm.at[idx_v.at[s]], rows_v.at[s], sg[s])

    def wait_gather(s):
        pltpu.make_async_copy(table_hbm.at[idx_v.at[s]], rows_v.at[s], sg[s]).wait()

    def start_write(g, s):
        for b in range(CB):
            pltpu.async_copy(
                rows_v.at[s, pl.ds(b * HIST, HIST), :],
                out_hbm.at[b_base + g * CB + b, pl.ds(0, HIST), pl.ds(0, EMBED_DIM)],
                sw[s],
            )

    def wait_write(g, s):
        for b in range(CB):
            pltpu.make_async_copy(
                rows_v.at[s, pl.ds(b * HIST, HIST), :],
                out_hbm.at[b_base + g * CB + b, pl.ds(0, HIST), pl.ds(0, EMBED_DIM)],
                sw[s],
            ).wait()

    load_idx(0, 0)
    start_gather(0)
    load_idx(1, 1)
    start_gather(1)
    wait_gather(0)
    start_write(0, 0)

    def pair_step(gp, _):
        for b in (0, 1):
            g = 2 * gp + b
            s = b
            wait_write(g - 2, s)
            load_idx(g, s)
            start_gather(s)
            wait_gather(1 - s)
            start_write(g - 1, 1 - s)
        return ()

    lax.fori_loop(1, N_CHUNKS // 2, pair_step, ())

    wait_gather(1)
    wait_write(N_CHUNKS - 2, 0)
    start_write(N_CHUNKS - 1, 1)
    wait_write(N_CHUNKS - 1, 1)


DENSE_CHUNK = 800
N_DENSE_CHUNKS = VOCAB // DENSE_CHUNK
DENSE_SR = DENSE_CHUNK * EMBED_DIM // 128


def _densify_body(table_hbm, dense_hbm, buf32_v, buf128_v, sem):
    wid = lax.axis_index("s") * NC + lax.axis_index("c")

    def step(g, _):
        cid = g * NW + wid

        @pl.when(cid < N_DENSE_CHUNKS)
        def _():
            r0 = pl.multiple_of(cid * DENSE_CHUNK, DENSE_CHUNK)
            sr0 = pl.multiple_of(cid * DENSE_SR, DENSE_SR)
            pltpu.sync_copy(table_hbm.at[pl.ds(r0, DENSE_CHUNK), :], buf32_v)

            def repack(t, _):
                for rr in range(4):
                    for c in range(2):
                        buf128_v[t, pl.ds(32 * rr + 16 * c, 16)] = (
                            buf32_v[4 * t + rr, pl.ds(16 * c, 16)]
                        )
                return ()

            lax.fori_loop(0, DENSE_SR, repack, ())
            pltpu.sync_copy(buf128_v, dense_hbm.at[pl.ds(sr0, DENSE_SR), :])

        return ()

    lax.fori_loop(0, (N_DENSE_CHUNKS + NW - 1) // NW, step, ())


def _densify(table):
    mesh = plsc.VectorSubcoreMesh(core_axis_name="c", subcore_axis_name="s")
    dense = pl.kernel(
        _densify_body,
        out_type=jax.ShapeDtypeStruct((VOCAB * EMBED_DIM // 128, 128), jnp.float32),
        mesh=mesh,
        scratch_types=[
            pltpu.VMEM((DENSE_CHUNK, EMBED_DIM), jnp.float32),
            pltpu.VMEM((DENSE_SR, 128), jnp.float32),
            pltpu.SemaphoreType.DMA,
        ],
    )(table)
    return dense.reshape(VOCAB, EMBED_DIM)


@jax.jit
def kernel(x, table):
    idx = x.reshape(-1).astype(jnp.int32)
    table = _densify(table)
    mesh = plsc.VectorSubcoreMesh(core_axis_name="c", subcore_axis_name="s")
    out = pl.kernel(
        _gather_body,
        out_type=jax.ShapeDtypeStruct((BATCH, HIST_PAD, D_PAD), jnp.float32),
        mesh=mesh,
        scratch_types=[
            pltpu.VMEM((2, CHUNK), jnp.int32),
            pltpu.VMEM((2, CHUNK, EMBED_DIM), jnp.float32),
            pltpu.SemaphoreType.DMA,
            pltpu.SemaphoreType.DMA,
            pltpu.SemaphoreType.DMA,
            pltpu.SemaphoreType.DMA,
        ],
        compiler_params=pltpu.CompilerParams(use_tc_tiling_on_sc=False),
    )(idx, table)
    return lax.slice(out, (0, 0, 0), (BATCH, HIST, EMBED_DIM))


# speedup vs baseline: 1.0041x; 1.0041x over previous
import jax
import jax.numpy as jnp
from jax import lax
from jax.experimental import pallas as pl
from jax.experimental.pallas import tpu as pltpu
from jax.experimental.pallas import tpu_sc as plsc

VOCAB = 1000000
EMBED_DIM = 32
BATCH = 16384
HIST = 50

HIST_PAD = 56
D_PAD = 128

NC = 2
NS = 16
NW = NC * NS

B_TOTAL = BATCH * HIST
BATCH_PER_W = BATCH // NW
CB = 32
CHUNK = CB * HIST
N_CHUNKS = BATCH_PER_W // CB


def _gather_body(idx_hbm, table_hbm, out_hbm, idx_v, rows_v, sg0, sg1, sw0, sw1):
    wid = lax.axis_index("s") * NC + lax.axis_index("c")
    b_base = wid * BATCH_PER_W
    sg = (sg0, sg1)
    sw = (sw0, sw1)

    def load_idx(g, s):
        off = (b_base + g * CB) * HIST
        pltpu.sync_copy(idx_hbm.at[pl.ds(off, CHUNK)], idx_v.at[s])

    def start_gather(s):
        pltpu.async_copy(table_hbm.at[idx_v.at[s]], rows_v.at[s], sg[s])

    def wait_gather(s):
        pltpu.make_async_copy(table_hbm.at[idx_v.at[s]], rows_v.at[s], sg[s]).wait()

    def start_write(g, s):
        for b in range(CB):
            pltpu.async_copy(
                rows_v.at[s, pl.ds(b * HIST, HIST), :],
                out_hbm.at[b_base + g * CB + b, pl.ds(0, HIST), pl.ds(0, EMBED_DIM)],
                sw[s],
            )

    def wait_write(g, s):
        for b in range(CB):
            pltpu.make_async_copy(
                rows_v.at[s, pl.ds(b * HIST, HIST), :],
                out_hbm.at[b_base + g * CB + b, pl.ds(0, HIST), pl.ds(0, EMBED_DIM)],
                sw[s],
            ).wait()

    load_idx(0, 0)
    start_gather(0)
    load_idx(1, 1)
    start_gather(1)
    wait_gather(0)
    start_write(0, 0)

    def pair_step(gp, _):
        for b in (0, 1):
            g = 2 * gp + b
            s = b
            wait_write(g - 2, s)
            load_idx(g, s)
            start_gather(s)
            wait_gather(1 - s)
            start_write(g - 1, 1 - s)
        return ()

    lax.fori_loop(1, N_CHUNKS // 2, pair_step, ())

    wait_gather(1)
    wait_write(N_CHUNKS - 2, 0)
    start_write(N_CHUNKS - 1, 1)
    wait_write(N_CHUNKS - 1, 1)


DENSE_CHUNK = 800
N_DENSE_CHUNKS = VOCAB // DENSE_CHUNK
DENSE_SR = DENSE_CHUNK * EMBED_DIM // 128


def _densify_body(table_hbm, dense_hbm, buf32_v, buf128_v, sem):
    wid = lax.axis_index("s") * NC + lax.axis_index("c")

    def step(g, _):
        cid = g * NW + wid

        @pl.when(cid < N_DENSE_CHUNKS)
        def _():
            r0 = pl.multiple_of(cid * DENSE_CHUNK, DENSE_CHUNK)
            sr0 = pl.multiple_of(cid * DENSE_SR, DENSE_SR)
            pltpu.sync_copy(table_hbm.at[pl.ds(r0, DENSE_CHUNK), :], buf32_v)

            def repack(r, _):
                for c in range(2):
                    buf128_v[pl.ds(32 * r + 16 * c, 16)] = (
                        buf32_v[r, pl.ds(16 * c, 16)]
                    )
                return ()

            lax.fori_loop(0, DENSE_CHUNK, repack, (), unroll=8)
            pltpu.sync_copy(buf128_v,
                            dense_hbm.at[pl.ds(sr0 * 128,
                                               DENSE_CHUNK * EMBED_DIM)])

        return ()

    lax.fori_loop(0, (N_DENSE_CHUNKS + NW - 1) // NW, step, ())


def _densify(table):
    mesh = plsc.VectorSubcoreMesh(core_axis_name="c", subcore_axis_name="s")
    dense = pl.kernel(
        _densify_body,
        out_type=jax.ShapeDtypeStruct((VOCAB * EMBED_DIM,), jnp.float32),
        mesh=mesh,
        scratch_types=[
            pltpu.VMEM((DENSE_CHUNK, EMBED_DIM), jnp.float32),
            pltpu.VMEM((DENSE_CHUNK * EMBED_DIM,), jnp.float32),
            pltpu.SemaphoreType.DMA,
        ],
    )(table)
    return dense.reshape(VOCAB, EMBED_DIM)


@jax.jit
def kernel(x, table):
    idx = x.reshape(-1).astype(jnp.int32)
    table = _densify(table)
    mesh = plsc.VectorSubcoreMesh(core_axis_name="c", subcore_axis_name="s")
    out = pl.kernel(
        _gather_body,
        out_type=jax.ShapeDtypeStruct((BATCH, HIST_PAD, D_PAD), jnp.float32),
        mesh=mesh,
        scratch_types=[
            pltpu.VMEM((2, CHUNK), jnp.int32),
            pltpu.VMEM((2, CHUNK, EMBED_DIM), jnp.float32),
            pltpu.SemaphoreType.DMA,
            pltpu.SemaphoreType.DMA,
            pltpu.SemaphoreType.DMA,
            pltpu.SemaphoreType.DMA,
        ],
        compiler_params=pltpu.CompilerParams(use_tc_tiling_on_sc=False),
    )(idx, table)
    return lax.slice(out, (0, 0, 0), (BATCH, HIST, EMBED_DIM))


# speedup vs baseline: 1.3520x; 1.3465x over previous
import jax
import jax.numpy as jnp
from jax import lax
from jax.experimental import pallas as pl
from jax.experimental.pallas import tpu as pltpu
from jax.experimental.pallas import tpu_sc as plsc

VOCAB = 1000000
EMBED_DIM = 32
BATCH = 16384
HIST = 50

HIST_PAD = 56
D_PAD = 128

NC = 2
NS = 16
NW = NC * NS

B_TOTAL = BATCH * HIST
BATCH_PER_W = BATCH // NW
CB = 32
CHUNK = CB * HIST
N_CHUNKS = BATCH_PER_W // CB


def _gather_body(idx_hbm, table_hbm, out_hbm, idx_v, rows_v, sg0, sg1, sw0, sw1):
    wid = lax.axis_index("s") * NC + lax.axis_index("c")
    b_base = wid * BATCH_PER_W
    sg = (sg0, sg1)
    sw = (sw0, sw1)

    def load_idx(g, s):
        off = (b_base + g * CB) * HIST
        pltpu.sync_copy(idx_hbm.at[pl.ds(off, CHUNK)], idx_v.at[s])

    def start_gather(s):
        pltpu.async_copy(table_hbm.at[idx_v.at[s]], rows_v.at[s], sg[s])

    def wait_gather(s):
        pltpu.make_async_copy(table_hbm.at[idx_v.at[s]], rows_v.at[s], sg[s]).wait()

    def start_write(g, s):
        for b in range(CB):
            pltpu.async_copy(
                rows_v.at[s, pl.ds(b * HIST, HIST), :],
                out_hbm.at[b_base + g * CB + b, pl.ds(0, HIST), pl.ds(0, EMBED_DIM)],
                sw[s],
            )

    def wait_write(g, s):
        for b in range(CB):
            pltpu.make_async_copy(
                rows_v.at[s, pl.ds(b * HIST, HIST), :],
                out_hbm.at[b_base + g * CB + b, pl.ds(0, HIST), pl.ds(0, EMBED_DIM)],
                sw[s],
            ).wait()

    load_idx(0, 0)
    start_gather(0)
    load_idx(1, 1)
    start_gather(1)
    wait_gather(0)
    start_write(0, 0)

    def pair_step(gp, _):
        for b in (0, 1):
            g = 2 * gp + b
            s = b
            wait_write(g - 2, s)
            load_idx(g, s)
            start_gather(s)
            wait_gather(1 - s)
            start_write(g - 1, 1 - s)
        return ()

    lax.fori_loop(1, N_CHUNKS // 2, pair_step, ())

    wait_gather(1)
    wait_write(N_CHUNKS - 2, 0)
    start_write(N_CHUNKS - 1, 1)
    wait_write(N_CHUNKS - 1, 1)


@jax.jit
def kernel(x, table):
    idx = x.reshape(-1).astype(jnp.int32)
    mesh = plsc.VectorSubcoreMesh(core_axis_name="c", subcore_axis_name="s")
    out = pl.kernel(
        _gather_body,
        out_type=jax.ShapeDtypeStruct((BATCH, HIST_PAD, D_PAD), jnp.float32),
        mesh=mesh,
        scratch_types=[
            pltpu.VMEM((2, CHUNK), jnp.int32),
            pltpu.VMEM((2, CHUNK, EMBED_DIM), jnp.float32),
            pltpu.SemaphoreType.DMA,
            pltpu.SemaphoreType.DMA,
            pltpu.SemaphoreType.DMA,
            pltpu.SemaphoreType.DMA,
        ],
        compiler_params=pltpu.CompilerParams(
            use_tc_tiling_on_sc=False, needs_layout_passes=False
        ),
    )(idx, table)
    return lax.slice(out, (0, 0, 0), (BATCH, HIST, EMBED_DIM))
